# baseline (speedup 1.0000x reference)
import functools

import jax
import jax.numpy as jnp
from jax import lax
from jax.experimental import pallas as pl
from jax.experimental.pallas import tpu as pltpu
from jax.experimental.pallas import tpu_sc as plsc

_B, _T, _D = 64, 2048, 1024
_L = 16
_NW = 16
_RPW = _B // _NW
_H = _RPW // 2


def _build():
    mesh = plsc.VectorSubcoreMesh(
        core_axis_name="c", subcore_axis_name="s",
        num_cores=1, num_subcores=_NW)

    @functools.partial(
        pl.kernel,
        mesh=mesh,
        out_type=jax.ShapeDtypeStruct((_B, _D), jnp.float32),
        scratch_types=[
            pltpu.VMEM((_L,), jnp.int32),
            pltpu.VMEM((_L,), jnp.int32),
            pltpu.VMEM((_H, _D), jnp.float32),
            pltpu.VMEM((_H, _D), jnp.float32),
            pltpu.SemaphoreType.DMA,
            pltpu.SemaphoreType.DMA,
            pltpu.SemaphoreType.DMA,
            pltpu.SemaphoreType.DMA,
        ],
    )
    def k(table_hbm, seq_hbm, out_hbm, seq_v, idx_v, rows_a, rows_b, g0, g1, o0, o1):
        wid = lax.axis_index("s") + lax.axis_index("c")
        chunk = lax.shift_right_logical(wid, 2)
        quarter = wid & 3
        pltpu.sync_copy(seq_hbm.at[pl.ds(chunk * _L, _L)], seq_v)
        lane = lax.iota(jnp.int32, _L)
        f = (lane & 1) + lax.shift_left(lax.shift_right_logical(lane, 3), 1)
        p = jnp.broadcast_to(quarter * _RPW, (_L,)) + f
        s = seq_v[...].at[p].get(mode="promise_in_bounds")
        b = jnp.broadcast_to(chunk * _L, (_L,)) + p
        idx_v[...] = b * _T + s - 1
        base = wid * _RPW
        cg0 = pltpu.async_copy(table_hbm.at[idx_v.at[pl.ds(0, _H)]], rows_a, g0)
        cg1 = pltpu.async_copy(table_hbm.at[idx_v.at[pl.ds(8, _H)]], rows_b, g1)
        cg0.wait()
        co0 = pltpu.async_copy(rows_a, out_hbm.at[pl.ds(base, _H)], o0)
        cg1.wait()
        co1 = pltpu.async_copy(rows_b, out_hbm.at[pl.ds(base + _H, _H)], o1)
        co0.wait()
        co1.wait()

    return k


_gather_last = _build()


def kernel(logits, seq_lens):
    B, T, D = logits.shape
    table = logits.reshape(B * T, D)
    out = _gather_last(table, seq_lens)
    return out
